# baseline (speedup 1.0000x reference)
import jax
import jax.numpy as jnp
from jax import lax
from jax.experimental import pallas as pl
from jax.experimental.pallas import tpu as pltpu
from jax.experimental.pallas import tpu_sc as plsc

NC, NS, L = 2, 16, 16
NW = NC * NS
B = 16384
D = 32
BPW = B // NW
CHUNK = 64
NCHUNK = BPW // CHUNK
GR_PER_FEAT = 62500
NSTREAM = (CHUNK * D) // 128


def _body(uidx_hbm, iidx_hbm, utT_hbm, itT_hbm, gb_hbm, out_hbm,
          uidx_v, iidx_v, ugr_v, igr_v, uix_v, iix_v, gb_v, out_v, sem):
    wid = lax.axis_index("s") * NC + lax.axis_index("c")
    pltpu.sync_copy(uidx_hbm.at[wid], uidx_v)
    pltpu.sync_copy(iidx_hbm.at[wid], iidx_v)
    pltpu.sync_copy(gb_hbm, gb_v)


    gb = gb_v[...]
    lane = lax.iota(jnp.int32, L)

    def chunk_fn(ch, _):
        base = ch * CHUNK

        def build(g, _):
            wu = uidx_v[pl.ds(base + g * L, L)] >> 4
            wi = iidx_v[pl.ds(base + g * L, L)] >> 4
            for c in range(D):
                off = c * CHUNK + g * L
                uix_v[pl.ds(off, L)] = wu + (c * GR_PER_FEAT)
                iix_v[pl.ds(off, L)] = wi + (c * GR_PER_FEAT)
            return 0

        lax.fori_loop(0, CHUNK // L, build, 0)

        copies = []
        for j in range(NSTREAM):
            sl = pl.ds(j * 128, 128)
            copies.append(pltpu.async_copy(utT_hbm.at[uix_v.at[sl]],
                                           ugr_v.at[sl], sem))
            copies.append(pltpu.async_copy(itT_hbm.at[iix_v.at[sl]],
                                           igr_v.at[sl], sem))
        for cp in copies:
            cp.wait()

        def extract(g, _):
            lane_u = uidx_v[pl.ds(base + g * L, L)] & 15
            lane_i = iidx_v[pl.ds(base + g * L, L)] & 15
            acc = jnp.zeros((L,), jnp.float32)
            for c in range(D):
                rows = jnp.full((L,), c * CHUNK + g * L, jnp.int32) + lane
                u = plsc.load_gather(ugr_v, [rows, lane_u])
                i = plsc.load_gather(igr_v, [rows, lane_i])
                acc = acc + u * i
            pred = acc + gb
            out_v[pl.ds(base + g * L, L)] = 1.0 / (1.0 + jnp.exp(-pred))
            return 0

        lax.fori_loop(0, CHUNK // L, extract, 0)
        return 0

    lax.fori_loop(0, NCHUNK, chunk_fn, 0)
    pltpu.sync_copy(out_v, out_hbm.at[wid])


@jax.jit
def _call(user_idx, item_idx, utT, itT, gb16):
    mesh = plsc.VectorSubcoreMesh(core_axis_name="c", subcore_axis_name="s",
                                  num_cores=NC, num_subcores=NS)
    fn = pl.kernel(
        _body,
        out_type=jax.ShapeDtypeStruct((NW, BPW), jnp.float32),
        mesh=mesh,
        compiler_params=pltpu.CompilerParams(needs_layout_passes=False,
                                             use_tc_tiling_on_sc=False),
        scratch_types=[
            pltpu.VMEM((BPW,), jnp.int32),
            pltpu.VMEM((BPW,), jnp.int32),
            pltpu.VMEM((CHUNK * D, 16), jnp.float32),
            pltpu.VMEM((CHUNK * D, 16), jnp.float32),
            pltpu.VMEM((CHUNK * D,), jnp.int32),
            pltpu.VMEM((CHUNK * D,), jnp.int32),
            pltpu.VMEM((L,), jnp.float32),
            pltpu.VMEM((BPW,), jnp.float32),
            pltpu.SemaphoreType.DMA,
        ],
    )
    out = fn(user_idx.reshape(NW, BPW), item_idx.reshape(NW, BPW),
             utT, itT, gb16)
    return out.reshape(B)


def kernel(user_idx, item_idx, user_table, item_table,
           user_bias_table, item_bias_table, global_bias):
    gb16 = jnp.broadcast_to(global_bias.astype(jnp.float32), (L,))
    return _call(user_idx.astype(jnp.int32), item_idx.astype(jnp.int32),
                 user_table.T.reshape(D * GR_PER_FEAT, 16),
                 item_table.T.reshape(D * GR_PER_FEAT, 16), gb16)


# speedup vs baseline: 5.5922x; 5.5922x over previous
import jax
import jax.numpy as jnp
from jax import lax
from jax.experimental import pallas as pl
from jax.experimental.pallas import tpu as pltpu
from jax.experimental.pallas import tpu_sc as plsc

NC, NS, L = 2, 16, 16
NW = NC * NS
B = 16384
D = 32
BPW = B // NW
CHUNK = 64
NCHUNK = BPW // CHUNK
QROWS = 250000
NSTREAM = 1


def _body(uidx_hbm, iidx_hbm, utT_hbm, itT_hbm, gb_hbm, out_hbm,
          uidx_v, iidx_v, ugr_v, igr_v, uix_v, iix_v, gb_v, out_v, sem):
    wid = lax.axis_index("s") * NC + lax.axis_index("c")
    pltpu.sync_copy(uidx_hbm.at[wid], uidx_v)
    pltpu.sync_copy(iidx_hbm.at[wid], iidx_v)
    pltpu.sync_copy(gb_hbm, gb_v)


    gb = gb_v[...]
    lane = lax.iota(jnp.int32, L)

    def chunk_fn(ch, _):
        base = ch * CHUNK

        def build(g, _):
            off = g * L
            uix_v[pl.ds(off, L)] = uidx_v[pl.ds(base + off, L)] >> 2
            iix_v[pl.ds(off, L)] = iidx_v[pl.ds(base + off, L)] >> 2
            return 0

        lax.fori_loop(0, CHUNK // L, build, 0)

        cu = pltpu.async_copy(utT_hbm.at[uix_v], ugr_v, sem)
        ci = pltpu.async_copy(itT_hbm.at[iix_v], igr_v, sem)
        cu.wait()
        ci.wait()

        def extract(g, _):
            cu0 = (uidx_v[pl.ds(base + g * L, L)] & 3) * D
            ci0 = (iidx_v[pl.ds(base + g * L, L)] & 3) * D
            rows = jnp.full((L,), g * L, jnp.int32) + lane
            acc = jnp.zeros((L,), jnp.float32)
            for c in range(D):
                u = plsc.load_gather(ugr_v, [rows, cu0 + c])
                i = plsc.load_gather(igr_v, [rows, ci0 + c])
                acc = acc + u * i
            pred = acc + gb
            out_v[pl.ds(base + g * L, L)] = 1.0 / (1.0 + jnp.exp(-pred))
            return 0

        lax.fori_loop(0, CHUNK // L, extract, 0)
        return 0

    lax.fori_loop(0, NCHUNK, chunk_fn, 0)
    pltpu.sync_copy(out_v, out_hbm.at[wid])


@jax.jit
def _call(user_idx, item_idx, utT, itT, gb16):
    mesh = plsc.VectorSubcoreMesh(core_axis_name="c", subcore_axis_name="s",
                                  num_cores=NC, num_subcores=NS)
    fn = pl.kernel(
        _body,
        out_type=jax.ShapeDtypeStruct((NW, BPW), jnp.float32),
        mesh=mesh,
        compiler_params=pltpu.CompilerParams(needs_layout_passes=False,
                                             use_tc_tiling_on_sc=False),
        scratch_types=[
            pltpu.VMEM((BPW,), jnp.int32),
            pltpu.VMEM((BPW,), jnp.int32),
            pltpu.VMEM((CHUNK, 128), jnp.float32),
            pltpu.VMEM((CHUNK, 128), jnp.float32),
            pltpu.VMEM((CHUNK,), jnp.int32),
            pltpu.VMEM((CHUNK,), jnp.int32),
            pltpu.VMEM((L,), jnp.float32),
            pltpu.VMEM((BPW,), jnp.float32),
            pltpu.SemaphoreType.DMA,
        ],
    )
    out = fn(user_idx.reshape(NW, BPW), item_idx.reshape(NW, BPW),
             utT, itT, gb16)
    return out.reshape(B)


def kernel(user_idx, item_idx, user_table, item_table,
           user_bias_table, item_bias_table, global_bias):
    gb16 = jnp.broadcast_to(global_bias.astype(jnp.float32), (L,))
    return _call(user_idx.astype(jnp.int32), item_idx.astype(jnp.int32),
                 user_table.reshape(QROWS, 128),
                 item_table.reshape(QROWS, 128), gb16)
